# baseline (speedup 1.0000x reference)
import functools

import jax
import jax.numpy as jnp
from jax import lax
from jax.experimental import pallas as pl
from jax.experimental.pallas import tpu as pltpu
from jax.experimental.pallas import tpu_sc as plsc

EPS = 1e-15
_CLIP = 1.0 - 1e-6


def _logmap0(x):
    norm = jnp.maximum(jnp.sqrt(jnp.sum(x * x, axis=-1, keepdims=True)), EPS)
    arg = jnp.clip(norm, 0.0, _CLIP)
    atanh = 0.5 * jnp.log((1.0 + arg) / (1.0 - arg))
    return x * atanh / norm


def _expmap0(u):
    norm = jnp.maximum(jnp.sqrt(jnp.sum(u * u, axis=-1, keepdims=True)), EPS)
    return jnp.tanh(norm) * u / norm


def _dense_layer_body(x_ref, w_ref, b_ref, o_ref):
    x = x_ref[...]
    t = _logmap0(x)
    h = lax.dot_general(t, w_ref[...], (((1,), (1,)), ((), ())),
                        preferred_element_type=jnp.float32) + b_ref[...]
    o_ref[...] = _expmap0(h)


def _dense_layer_mid_body(p0_ref, p1_ref, w_ref, b_ref, o_ref):
    x = jnp.maximum(p0_ref[0] + p1_ref[0], 0.0)
    t = _logmap0(x)
    h = lax.dot_general(t, w_ref[...], (((1,), (1,)), ((), ())),
                        preferred_element_type=jnp.float32) + b_ref[...]
    o_ref[...] = _expmap0(h)


def _classifier_body(p0_ref, p1_ref, w_ref, b_ref, o_ref):
    x = jnp.maximum(p0_ref[0] + p1_ref[0], 0.0)
    t = _logmap0(x)
    o_ref[...] = lax.dot_general(t, w_ref[...], (((1,), (1,)), ((), ())),
                                 preferred_element_type=jnp.float32) + b_ref[...]


def _dense_first(x, W, b):
    n, d = x.shape
    blk = 2000
    grid = n // blk
    return pl.pallas_call(
        _dense_layer_body,
        grid=(grid,),
        in_specs=[
            pl.BlockSpec((blk, d), lambda i: (i, 0)),
            pl.BlockSpec((d, d), lambda i: (0, 0)),
            pl.BlockSpec((1, d), lambda i: (0, 0)),
        ],
        out_specs=pl.BlockSpec((blk, d), lambda i: (i, 0)),
        out_shape=jax.ShapeDtypeStruct((n, d), jnp.float32),
    )(x, W, b.reshape(1, d))


def _dense_mid(partials, W, b, n):
    d = partials.shape[2]
    blk = 2000
    grid = n // blk
    return pl.pallas_call(
        _dense_layer_mid_body,
        grid=(grid,),
        in_specs=[
            pl.BlockSpec((1, blk, d), lambda i: (0, i, 0)),
            pl.BlockSpec((1, blk, d), lambda i: (1, i, 0)),
            pl.BlockSpec((d, d), lambda i: (0, 0)),
            pl.BlockSpec((1, d), lambda i: (0, 0)),
        ],
        out_specs=pl.BlockSpec((blk, d), lambda i: (i, 0)),
        out_shape=jax.ShapeDtypeStruct((n, d), jnp.float32),
    )(partials, partials, W, b.reshape(1, d))


def _classifier(partials, Wc, bc, n):
    d = partials.shape[2]
    nc = Wc.shape[0]
    ncp = 16
    Wp = jnp.zeros((ncp, d), jnp.float32).at[:nc].set(Wc)
    bp = jnp.zeros((ncp,), jnp.float32).at[:nc].set(bc)
    blk = 2000
    grid = n // blk
    out = pl.pallas_call(
        _classifier_body,
        grid=(grid,),
        in_specs=[
            pl.BlockSpec((1, blk, d), lambda i: (0, i, 0)),
            pl.BlockSpec((1, blk, d), lambda i: (1, i, 0)),
            pl.BlockSpec((ncp, d), lambda i: (0, 0)),
            pl.BlockSpec((1, ncp), lambda i: (0, 0)),
        ],
        out_specs=pl.BlockSpec((blk, ncp), lambda i: (i, 0)),
        out_shape=jax.ShapeDtypeStruct((n, ncp), jnp.float32),
    )(partials, partials, Wp, bp.reshape(1, ncp))
    return out[:, :nc]


def _scatter_partials(y, src, dst):
    n, d = y.shape
    e = src.shape[0]
    NC, NS = 2, 16
    NW = NC * NS
    K = 64
    steps = 157
    epw = steps * K
    NP = 10240
    rpt = NP // NS

    ppw = epw - e // NW
    src_p = jnp.concatenate(
        [src.reshape(NW, e // NW), jnp.zeros((NW, ppw), jnp.int32)], axis=1
    ).reshape(NW * steps * K)
    pad_dst = jnp.broadcast_to(n + jnp.arange(ppw, dtype=jnp.int32), (NW, ppw))
    dst_p = jnp.concatenate(
        [dst.reshape(NW, e // NW), pad_dst], axis=1
    ).reshape(NW * steps * K)

    mesh = plsc.VectorSubcoreMesh(core_axis_name="c", subcore_axis_name="s")

    @functools.partial(
        pl.kernel,
        mesh=mesh,
        out_type=jax.ShapeDtypeStruct((NC, NP, d), jnp.float32),
        scratch_types=[
            pltpu.VMEM((K,), jnp.int32),
            pltpu.VMEM((K,), jnp.int32),
            pltpu.VMEM((K,), jnp.int32),
            pltpu.VMEM((K,), jnp.int32),
            pltpu.VMEM((K, d), jnp.float32),
            pltpu.VMEM((K, d), jnp.float32),
            pltpu.VMEM_SHARED((NP, d), jnp.float32),
            pltpu.SemaphoreType.DMA,
            pltpu.SemaphoreType.DMA,
        ],
    )
    def k(src_hbm, dst_hbm, y_hbm, out_hbm, sv0, sv1, dv0, dv1, rows0, rows1,
          acc, g0, g1):
        c = lax.axis_index("c")
        s = lax.axis_index("s")
        wid = c * NS + s
        base0 = wid * epw

        def zrow(i, carry):
            for j in range(d // 16):
                rows0[i, pl.ds(j * 16, 16)] = jnp.zeros((16,), jnp.float32)
            return carry

        lax.fori_loop(0, K, zrow, 0)
        for r in range(rpt // K):
            pltpu.sync_copy(rows0, acc.at[pl.ds(s * rpt + r * K, K)])
        plsc.subcore_barrier()

        pltpu.sync_copy(src_hbm.at[pl.ds(base0, K)], sv0)
        pltpu.sync_copy(dst_hbm.at[pl.ds(base0, K)], dv0)
        pltpu.async_copy(y_hbm.at[sv0], rows0, g0)
        pltpu.sync_copy(src_hbm.at[pl.ds(base0 + K, K)], sv1)
        pltpu.sync_copy(dst_hbm.at[pl.ds(base0 + K, K)], dv1)
        pltpu.async_copy(y_hbm.at[sv1], rows1, g1)

        def half(cn, sv, dv, rows, g):
            pltpu.make_async_copy(y_hbm.at[sv], rows, g).wait()
            pltpu.sync_copy(rows, acc.at[dv], add=True)

            @pl.when(cn < steps)
            def _():
                base = base0 + cn * K
                pltpu.sync_copy(src_hbm.at[pl.ds(base, K)], sv)
                pltpu.sync_copy(dst_hbm.at[pl.ds(base, K)], dv)
                pltpu.async_copy(y_hbm.at[sv], rows, g)

        def step(j, carry):
            c0 = 2 * j
            half(c0 + 2, sv0, dv0, rows0, g0)
            half(c0 + 3, sv1, dv1, rows1, g1)
            return carry

        lax.fori_loop(0, steps // 2, step, 0)
        if steps % 2:
            pltpu.make_async_copy(y_hbm.at[sv0], rows0, g0).wait()
            pltpu.sync_copy(rows0, acc.at[dv0], add=True)
        plsc.subcore_barrier()
        pltpu.sync_copy(acc.at[pl.ds(s * rpt, rpt)],
                        out_hbm.at[c, pl.ds(s * rpt, rpt)])

    return k(src_p, dst_p, y)


def kernel(edge_index, entity_embeddings, W1, b1, W2, b2, Wc, bc):
    src = edge_index[0]
    dst = edge_index[1]
    n = entity_embeddings.shape[0]
    y1 = _dense_first(entity_embeddings, W1, b1)
    p1 = _scatter_partials(y1, src, dst)
    y2 = _dense_mid(p1, W2, b2, n)
    p2 = _scatter_partials(y2, src, dst)
    return _classifier(p2, Wc, bc, n)


# speedup vs baseline: 1.3641x; 1.3641x over previous
import functools

import jax
import jax.numpy as jnp
from jax import lax
from jax.experimental import pallas as pl
from jax.experimental.pallas import tpu as pltpu
from jax.experimental.pallas import tpu_sc as plsc

EPS = 1e-15
_CLIP = 1.0 - 1e-6


def _logmap0(x):
    norm = jnp.maximum(jnp.sqrt(jnp.sum(x * x, axis=-1, keepdims=True)), EPS)
    arg = jnp.clip(norm, 0.0, _CLIP)
    atanh = 0.5 * jnp.log((1.0 + arg) / (1.0 - arg))
    return x * atanh / norm


def _expmap0(u):
    norm = jnp.maximum(jnp.sqrt(jnp.sum(u * u, axis=-1, keepdims=True)), EPS)
    return jnp.tanh(norm) * u / norm


def _dense_layer_body(x_ref, w_ref, b_ref, o_ref):
    x = x_ref[...]
    t = _logmap0(x)
    h = lax.dot_general(t, w_ref[...], (((1,), (1,)), ((), ())),
                        preferred_element_type=jnp.float32) + b_ref[...]
    o_ref[...] = _expmap0(h)


def _dense_layer_mid_body(p0_ref, p1_ref, w_ref, b_ref, o_ref):
    x = jnp.maximum(p0_ref[0] + p1_ref[0], 0.0)
    t = _logmap0(x)
    h = lax.dot_general(t, w_ref[...], (((1,), (1,)), ((), ())),
                        preferred_element_type=jnp.float32) + b_ref[...]
    o_ref[...] = _expmap0(h)


def _classifier_body(p0_ref, p1_ref, w_ref, b_ref, o_ref):
    x = jnp.maximum(p0_ref[0] + p1_ref[0], 0.0)
    t = _logmap0(x)
    o_ref[...] = lax.dot_general(t, w_ref[...], (((1,), (1,)), ((), ())),
                                 preferred_element_type=jnp.float32) + b_ref[...]


def _dense_first(x, W, b):
    n, d = x.shape
    blk = 2000
    grid = n // blk
    return pl.pallas_call(
        _dense_layer_body,
        grid=(grid,),
        in_specs=[
            pl.BlockSpec((blk, d), lambda i: (i, 0)),
            pl.BlockSpec((d, d), lambda i: (0, 0)),
            pl.BlockSpec((1, d), lambda i: (0, 0)),
        ],
        out_specs=pl.BlockSpec((blk, d), lambda i: (i, 0)),
        out_shape=jax.ShapeDtypeStruct((n, d), jnp.float32),
    )(x, W, b.reshape(1, d))


def _dense_mid(partials, W, b, n):
    d = partials.shape[2]
    blk = 2000
    grid = n // blk
    return pl.pallas_call(
        _dense_layer_mid_body,
        grid=(grid,),
        in_specs=[
            pl.BlockSpec((1, blk, d), lambda i: (0, i, 0)),
            pl.BlockSpec((1, blk, d), lambda i: (1, i, 0)),
            pl.BlockSpec((d, d), lambda i: (0, 0)),
            pl.BlockSpec((1, d), lambda i: (0, 0)),
        ],
        out_specs=pl.BlockSpec((blk, d), lambda i: (i, 0)),
        out_shape=jax.ShapeDtypeStruct((n, d), jnp.float32),
    )(partials, partials, W, b.reshape(1, d))


def _classifier(partials, Wc, bc, n):
    d = partials.shape[2]
    nc = Wc.shape[0]
    ncp = 16
    Wp = jnp.zeros((ncp, d), jnp.float32).at[:nc].set(Wc)
    bp = jnp.zeros((ncp,), jnp.float32).at[:nc].set(bc)
    blk = 2000
    grid = n // blk
    out = pl.pallas_call(
        _classifier_body,
        grid=(grid,),
        in_specs=[
            pl.BlockSpec((1, blk, d), lambda i: (0, i, 0)),
            pl.BlockSpec((1, blk, d), lambda i: (1, i, 0)),
            pl.BlockSpec((ncp, d), lambda i: (0, 0)),
            pl.BlockSpec((1, ncp), lambda i: (0, 0)),
        ],
        out_specs=pl.BlockSpec((blk, ncp), lambda i: (i, 0)),
        out_shape=jax.ShapeDtypeStruct((n, ncp), jnp.float32),
    )(partials, partials, Wp, bp.reshape(1, ncp))
    return out[:, :nc]


def _scatter_partials(y, src, dst):
    n, d = y.shape
    e = src.shape[0]
    NC, NS = 2, 16
    NW = NC * NS
    K = 80
    steps = 125
    epw = steps * K
    NP = 10240
    rpt = NP // NS

    ppw = epw - e // NW
    src_p = jnp.concatenate(
        [src.reshape(NW, e // NW), jnp.zeros((NW, ppw), jnp.int32)], axis=1
    ).reshape(NW * steps * K)
    pad_dst = jnp.broadcast_to(n + jnp.arange(ppw, dtype=jnp.int32), (NW, ppw))
    dst_p = jnp.concatenate(
        [dst.reshape(NW, e // NW), pad_dst], axis=1
    ).reshape(NW * steps * K)

    mesh = plsc.VectorSubcoreMesh(core_axis_name="c", subcore_axis_name="s")

    @functools.partial(
        pl.kernel,
        mesh=mesh,
        out_type=jax.ShapeDtypeStruct((NC, NP, d), jnp.float32),
        scratch_types=[
            pltpu.VMEM((K,), jnp.int32),
            pltpu.VMEM((K,), jnp.int32),
            pltpu.VMEM((K,), jnp.int32),
            pltpu.VMEM((K,), jnp.int32),
            pltpu.VMEM((K, d), jnp.float32),
            pltpu.VMEM((K, d), jnp.float32),
            pltpu.VMEM_SHARED((NP, d), jnp.float32),
            pltpu.SemaphoreType.DMA,
            pltpu.SemaphoreType.DMA,
        ],
    )
    def k(src_hbm, dst_hbm, y_hbm, out_hbm, sv0, sv1, dv0, dv1, rows0, rows1,
          acc, g0, g1):
        c = lax.axis_index("c")
        s = lax.axis_index("s")
        wid = c * NS + s
        base0 = wid * epw

        def zrow(i, carry):
            for j in range(d // 16):
                rows0[i, pl.ds(j * 16, 16)] = jnp.zeros((16,), jnp.float32)
            return carry

        lax.fori_loop(0, K, zrow, 0)
        for r in range(rpt // K):
            pltpu.sync_copy(rows0, acc.at[pl.ds(s * rpt + r * K, K)])
        plsc.subcore_barrier()

        pltpu.sync_copy(src_hbm.at[pl.ds(base0, K)], sv0)
        pltpu.sync_copy(dst_hbm.at[pl.ds(base0, K)], dv0)
        pltpu.async_copy(y_hbm.at[sv0], rows0, g0)
        pltpu.sync_copy(src_hbm.at[pl.ds(base0 + K, K)], sv1)
        pltpu.sync_copy(dst_hbm.at[pl.ds(base0 + K, K)], dv1)
        pltpu.async_copy(y_hbm.at[sv1], rows1, g1)

        def half(cn, sv, dv, rows, g):
            pltpu.make_async_copy(y_hbm.at[sv], rows, g).wait()
            pltpu.sync_copy(rows, acc.at[dv], add=True)

            @pl.when(cn < steps)
            def _():
                base = base0 + cn * K
                pltpu.sync_copy(src_hbm.at[pl.ds(base, K)], sv)
                pltpu.sync_copy(dst_hbm.at[pl.ds(base, K)], dv)
                pltpu.async_copy(y_hbm.at[sv], rows, g)

        def step(j, carry):
            c0 = 2 * j
            half(c0 + 2, sv0, dv0, rows0, g0)
            half(c0 + 3, sv1, dv1, rows1, g1)
            return carry

        lax.fori_loop(0, steps // 2, step, 0)
        if steps % 2:
            pltpu.make_async_copy(y_hbm.at[sv0], rows0, g0).wait()
            pltpu.sync_copy(rows0, acc.at[dv0], add=True)
        plsc.subcore_barrier()
        pltpu.sync_copy(acc.at[pl.ds(s * rpt, rpt)],
                        out_hbm.at[c, pl.ds(s * rpt, rpt)])

    return k(src_p, dst_p, y)


def kernel(edge_index, entity_embeddings, W1, b1, W2, b2, Wc, bc):
    src = edge_index[0]
    dst = edge_index[1]
    n = entity_embeddings.shape[0]
    y1 = _dense_first(entity_embeddings, W1, b1)
    p1 = _scatter_partials(y1, src, dst)
    y2 = _dense_mid(p1, W2, b2, n)
    p2 = _scatter_partials(y2, src, dst)
    return _classifier(p2, Wc, bc, n)


# speedup vs baseline: 1.9281x; 1.4135x over previous
import functools

import jax
import jax.numpy as jnp
from jax import lax
from jax.experimental import pallas as pl
from jax.experimental.pallas import tpu as pltpu
from jax.experimental.pallas import tpu_sc as plsc

EPS = 1e-15
_CLIP = 1.0 - 1e-6


def _logmap0(x):
    norm = jnp.maximum(jnp.sqrt(jnp.sum(x * x, axis=-1, keepdims=True)), EPS)
    arg = jnp.clip(norm, 0.0, _CLIP)
    atanh = 0.5 * jnp.log((1.0 + arg) / (1.0 - arg))
    return x * atanh / norm


def _expmap0(u):
    norm = jnp.maximum(jnp.sqrt(jnp.sum(u * u, axis=-1, keepdims=True)), EPS)
    return jnp.tanh(norm) * u / norm


def _dense_layer_body(x_ref, w_ref, b_ref, o_ref):
    x = x_ref[...]
    t = _logmap0(x)
    h = lax.dot_general(t, w_ref[...], (((1,), (1,)), ((), ())),
                        preferred_element_type=jnp.float32) + b_ref[...]
    o_ref[...] = _expmap0(h)


def _dense_layer_mid_body(p0_ref, p1_ref, w_ref, b_ref, o_ref):
    x = jnp.maximum(p0_ref[0] + p1_ref[0], 0.0)
    t = _logmap0(x)
    h = lax.dot_general(t, w_ref[...], (((1,), (1,)), ((), ())),
                        preferred_element_type=jnp.float32) + b_ref[...]
    o_ref[...] = _expmap0(h)


def _classifier_body(p0_ref, p1_ref, w_ref, b_ref, o_ref):
    x = jnp.maximum(p0_ref[0] + p1_ref[0], 0.0)
    t = _logmap0(x)
    o_ref[...] = lax.dot_general(t, w_ref[...], (((1,), (1,)), ((), ())),
                                 preferred_element_type=jnp.float32) + b_ref[...]


def _dense_first(x, W, b):
    n, d = x.shape
    blk = 2000
    grid = n // blk
    return pl.pallas_call(
        _dense_layer_body,
        grid=(grid,),
        in_specs=[
            pl.BlockSpec((blk, d), lambda i: (i, 0)),
            pl.BlockSpec((d, d), lambda i: (0, 0)),
            pl.BlockSpec((1, d), lambda i: (0, 0)),
        ],
        out_specs=pl.BlockSpec((blk, d), lambda i: (i, 0)),
        out_shape=jax.ShapeDtypeStruct((n, d), jnp.float32),
    )(x, W, b.reshape(1, d))


def _dense_mid(partials, W, b, n):
    d = partials.shape[2]
    blk = 2000
    grid = n // blk
    return pl.pallas_call(
        _dense_layer_mid_body,
        grid=(grid,),
        in_specs=[
            pl.BlockSpec((1, blk, d), lambda i: (0, i, 0)),
            pl.BlockSpec((1, blk, d), lambda i: (1, i, 0)),
            pl.BlockSpec((d, d), lambda i: (0, 0)),
            pl.BlockSpec((1, d), lambda i: (0, 0)),
        ],
        out_specs=pl.BlockSpec((blk, d), lambda i: (i, 0)),
        out_shape=jax.ShapeDtypeStruct((n, d), jnp.float32),
    )(partials, partials, W, b.reshape(1, d))


def _classifier(partials, Wc, bc, n):
    d = partials.shape[2]
    nc = Wc.shape[0]
    ncp = 16
    Wp = jnp.zeros((ncp, d), jnp.float32).at[:nc].set(Wc)
    bp = jnp.zeros((ncp,), jnp.float32).at[:nc].set(bc)
    blk = 2000
    grid = n // blk
    out = pl.pallas_call(
        _classifier_body,
        grid=(grid,),
        in_specs=[
            pl.BlockSpec((1, blk, d), lambda i: (0, i, 0)),
            pl.BlockSpec((1, blk, d), lambda i: (1, i, 0)),
            pl.BlockSpec((ncp, d), lambda i: (0, 0)),
            pl.BlockSpec((1, ncp), lambda i: (0, 0)),
        ],
        out_specs=pl.BlockSpec((blk, ncp), lambda i: (i, 0)),
        out_shape=jax.ShapeDtypeStruct((n, ncp), jnp.float32),
    )(partials, partials, Wp, bp.reshape(1, ncp))
    return out[:, :nc]


def _scatter_partials(y, src, dst):
    n, d = y.shape
    e = src.shape[0]
    NC, NS = 2, 16
    NW = NC * NS
    K = 80
    steps = 125
    epw = steps * K
    NP = 10240
    rpt = NP // NS

    ppw = epw - e // NW
    src_p = jnp.concatenate(
        [src.reshape(NW, e // NW), jnp.zeros((NW, ppw), jnp.int32)], axis=1
    ).reshape(NW * steps * K)
    pad_dst = jnp.broadcast_to(n + jnp.arange(ppw, dtype=jnp.int32), (NW, ppw))
    dst_p = jnp.concatenate(
        [dst.reshape(NW, e // NW), pad_dst], axis=1
    ).reshape(NW * steps * K)

    mesh = plsc.VectorSubcoreMesh(core_axis_name="c", subcore_axis_name="s")

    @functools.partial(
        pl.kernel,
        mesh=mesh,
        out_type=jax.ShapeDtypeStruct((NC, NP, d), jnp.float32),
        scratch_types=[
            pltpu.VMEM((K,), jnp.int32),
            pltpu.VMEM((K,), jnp.int32),
            pltpu.VMEM((K,), jnp.int32),
            pltpu.VMEM((K,), jnp.int32),
            pltpu.VMEM((K, d), jnp.float32),
            pltpu.VMEM((K, d), jnp.float32),
            pltpu.VMEM((K, d), jnp.float32),
            pltpu.VMEM_SHARED((NP, d), jnp.float32),
            pltpu.SemaphoreType.DMA,
            pltpu.SemaphoreType.DMA,
            pltpu.SemaphoreType.DMA,
            pltpu.SemaphoreType.DMA,
            pltpu.SemaphoreType.DMA,
            pltpu.SemaphoreType.DMA,
        ],
    )
    def k(src_hbm, dst_hbm, y_hbm, out_hbm, sv0, sv1, dv0, dv1, rows0, rows1,
          zbuf, acc, g0, g1, si0, si1, di0, di1):
        c = lax.axis_index("c")
        s = lax.axis_index("s")
        wid = c * NS + s
        base0 = wid * epw

        pltpu.async_copy(src_hbm.at[pl.ds(base0, K)], sv0, si0)
        pltpu.async_copy(dst_hbm.at[pl.ds(base0, K)], dv0, di0)
        pltpu.async_copy(src_hbm.at[pl.ds(base0 + K, K)], sv1, si1)
        pltpu.async_copy(dst_hbm.at[pl.ds(base0 + K, K)], dv1, di1)

        def zrow(i, carry):
            for j in range(d // 16):
                zbuf[i, pl.ds(j * 16, 16)] = jnp.zeros((16,), jnp.float32)
            return carry

        lax.fori_loop(0, K, zrow, 0)
        for r in range(rpt // K):
            pltpu.sync_copy(zbuf, acc.at[pl.ds(s * rpt + r * K, K)])
        plsc.subcore_barrier()

        pltpu.make_async_copy(src_hbm.at[pl.ds(base0, K)], sv0, si0).wait()
        pltpu.async_copy(y_hbm.at[sv0], rows0, g0)
        pltpu.make_async_copy(src_hbm.at[pl.ds(base0, K)], sv1, si1).wait()
        pltpu.async_copy(y_hbm.at[sv1], rows1, g1)

        def half(cn, sv, dv, rows, g, si, di):
            pltpu.make_async_copy(y_hbm.at[sv], rows, g).wait()
            base = base0 + cn * K

            @pl.when(cn < steps)
            def _():
                pltpu.async_copy(src_hbm.at[pl.ds(base, K)], sv, si)

            pltpu.make_async_copy(dst_hbm.at[pl.ds(base0, K)], dv, di).wait()
            pltpu.sync_copy(rows, acc.at[dv], add=True)

            @pl.when(cn < steps)
            def _():
                pltpu.async_copy(dst_hbm.at[pl.ds(base, K)], dv, di)
                pltpu.make_async_copy(src_hbm.at[pl.ds(base0, K)], sv, si).wait()
                pltpu.async_copy(y_hbm.at[sv], rows, g)

        def step(j, carry):
            c0 = 2 * j
            half(c0 + 2, sv0, dv0, rows0, g0, si0, di0)
            half(c0 + 3, sv1, dv1, rows1, g1, si1, di1)
            return carry

        lax.fori_loop(0, steps // 2, step, 0)
        if steps % 2:
            pltpu.make_async_copy(y_hbm.at[sv0], rows0, g0).wait()
            pltpu.make_async_copy(dst_hbm.at[pl.ds(base0, K)], dv0, di0).wait()
            pltpu.sync_copy(rows0, acc.at[dv0], add=True)
        plsc.subcore_barrier()
        pltpu.sync_copy(acc.at[pl.ds(s * rpt, rpt)],
                        out_hbm.at[c, pl.ds(s * rpt, rpt)])

    return k(src_p, dst_p, y)


def kernel(edge_index, entity_embeddings, W1, b1, W2, b2, Wc, bc):
    src = edge_index[0]
    dst = edge_index[1]
    n = entity_embeddings.shape[0]
    y1 = _dense_first(entity_embeddings, W1, b1)
    p1 = _scatter_partials(y1, src, dst)
    y2 = _dense_mid(p1, W2, b2, n)
    p2 = _scatter_partials(y2, src, dst)
    return _classifier(p2, Wc, bc, n)


# speedup vs baseline: 2.2400x; 1.1618x over previous
import functools

import jax
import jax.numpy as jnp
from jax import lax
from jax.experimental import pallas as pl
from jax.experimental.pallas import tpu as pltpu
from jax.experimental.pallas import tpu_sc as plsc

EPS = 1e-15
_CLIP = 1.0 - 1e-6


def _logmap0(x):
    norm = jnp.maximum(jnp.sqrt(jnp.sum(x * x, axis=-1, keepdims=True)), EPS)
    arg = jnp.clip(norm, 0.0, _CLIP)
    atanh = 0.5 * jnp.log((1.0 + arg) / (1.0 - arg))
    return x * atanh / norm


def _expmap0(u):
    norm = jnp.maximum(jnp.sqrt(jnp.sum(u * u, axis=-1, keepdims=True)), EPS)
    return jnp.tanh(norm) * u / norm


def _dense_layer_body(x_ref, w_ref, b_ref, o_ref):
    x = x_ref[...]
    t = _logmap0(x)
    h = lax.dot_general(t, w_ref[...], (((1,), (1,)), ((), ())),
                        preferred_element_type=jnp.float32) + b_ref[...]
    o_ref[...] = _expmap0(h)


def _dense_layer_mid_body(p0_ref, p1_ref, w_ref, b_ref, o_ref):
    x = jnp.maximum(p0_ref[0] + p1_ref[0], 0.0)
    t = _logmap0(x)
    h = lax.dot_general(t, w_ref[...], (((1,), (1,)), ((), ())),
                        preferred_element_type=jnp.float32) + b_ref[...]
    o_ref[...] = _expmap0(h)


def _classifier_body(p0_ref, p1_ref, w_ref, b_ref, o_ref):
    x = jnp.maximum(p0_ref[0] + p1_ref[0], 0.0)
    t = _logmap0(x)
    o_ref[...] = lax.dot_general(t, w_ref[...], (((1,), (1,)), ((), ())),
                                 preferred_element_type=jnp.float32) + b_ref[...]


def _dense_first(x, W, b):
    n, d = x.shape
    blk = 2000
    grid = n // blk
    return pl.pallas_call(
        _dense_layer_body,
        grid=(grid,),
        in_specs=[
            pl.BlockSpec((blk, d), lambda i: (i, 0)),
            pl.BlockSpec((d, d), lambda i: (0, 0)),
            pl.BlockSpec((1, d), lambda i: (0, 0)),
        ],
        out_specs=pl.BlockSpec((blk, d), lambda i: (i, 0)),
        out_shape=jax.ShapeDtypeStruct((n, d), jnp.float32),
    )(x, W, b.reshape(1, d))


def _dense_mid(partials, W, b, n):
    d = partials.shape[2]
    blk = 2000
    grid = n // blk
    return pl.pallas_call(
        _dense_layer_mid_body,
        grid=(grid,),
        in_specs=[
            pl.BlockSpec((1, blk, d), lambda i: (0, i, 0)),
            pl.BlockSpec((1, blk, d), lambda i: (1, i, 0)),
            pl.BlockSpec((d, d), lambda i: (0, 0)),
            pl.BlockSpec((1, d), lambda i: (0, 0)),
        ],
        out_specs=pl.BlockSpec((blk, d), lambda i: (i, 0)),
        out_shape=jax.ShapeDtypeStruct((n, d), jnp.float32),
    )(partials, partials, W, b.reshape(1, d))


def _classifier(partials, Wc, bc, n):
    d = partials.shape[2]
    nc = Wc.shape[0]
    ncp = 16
    Wp = jnp.zeros((ncp, d), jnp.float32).at[:nc].set(Wc)
    bp = jnp.zeros((ncp,), jnp.float32).at[:nc].set(bc)
    blk = 2000
    grid = n // blk
    out = pl.pallas_call(
        _classifier_body,
        grid=(grid,),
        in_specs=[
            pl.BlockSpec((1, blk, d), lambda i: (0, i, 0)),
            pl.BlockSpec((1, blk, d), lambda i: (1, i, 0)),
            pl.BlockSpec((ncp, d), lambda i: (0, 0)),
            pl.BlockSpec((1, ncp), lambda i: (0, 0)),
        ],
        out_specs=pl.BlockSpec((blk, ncp), lambda i: (i, 0)),
        out_shape=jax.ShapeDtypeStruct((n, ncp), jnp.float32),
    )(partials, partials, Wp, bp.reshape(1, ncp))
    return out[:, :nc]


def _scatter_partials(y, src, dst):
    n, d = y.shape
    e = src.shape[0]
    NC, NS = 2, 16
    NW = NC * NS
    K = 80
    steps = 125
    epw = steps * K
    NP = 10240
    rpt = NP // NS

    ppw = epw - e // NW
    src_p = jnp.concatenate(
        [src.reshape(NW, e // NW), jnp.zeros((NW, ppw), jnp.int32)], axis=1
    ).reshape(NW * steps * K)
    pad_dst = jnp.broadcast_to(n + jnp.arange(ppw, dtype=jnp.int32), (NW, ppw))
    dst_p = jnp.concatenate(
        [dst.reshape(NW, e // NW), pad_dst], axis=1
    ).reshape(NW * steps * K)

    mesh = plsc.VectorSubcoreMesh(core_axis_name="c", subcore_axis_name="s")

    @functools.partial(
        pl.kernel,
        mesh=mesh,
        out_type=jax.ShapeDtypeStruct((NC, NP, d), jnp.float32),
        scratch_types=(
            [pltpu.VMEM((K,), jnp.int32)] * 6
            + [pltpu.VMEM((K, d), jnp.float32)] * 3
            + [pltpu.VMEM_SHARED((NP, d), jnp.float32)]
            + [pltpu.SemaphoreType.DMA] * 12
        ),
    )
    def k(src_hbm, dst_hbm, y_hbm, out_hbm, sv0, sv1, sv2, dv0, dv1, dv2,
          rows0, rows1, rows2, acc, si0, si1, si2, di0, di1, di2,
          g0, g1, g2, ss0, ss1, ss2):
        c = lax.axis_index("c")
        s = lax.axis_index("s")
        wid = c * NS + s
        base0 = wid * epw
        svs, dvs = (sv0, sv1, sv2), (dv0, dv1, dv2)
        rbufs = (rows0, rows1, rows2)
        sis, dis = (si0, si1, si2), (di0, di1, di2)
        gs, sss = (g0, g1, g2), (ss0, ss1, ss2)

        def load_src(cn, p):
            pltpu.async_copy(src_hbm.at[pl.ds(base0 + cn * K, K)], svs[p], sis[p])

        def load_dst(cn, p):
            pltpu.async_copy(dst_hbm.at[pl.ds(base0 + cn * K, K)], dvs[p], dis[p])

        def wait_src(p):
            pltpu.make_async_copy(src_hbm.at[pl.ds(base0, K)], svs[p], sis[p]).wait()

        def wait_dst(p):
            pltpu.make_async_copy(dst_hbm.at[pl.ds(base0, K)], dvs[p], dis[p]).wait()

        def issue_gather(p):
            pltpu.async_copy(y_hbm.at[svs[p]], rbufs[p], gs[p])

        def wait_gather(p):
            pltpu.make_async_copy(y_hbm.at[svs[p]], rbufs[p], gs[p]).wait()

        def issue_scatter(p):
            pltpu.async_copy(rbufs[p], acc.at[dvs[p]], sss[p], add=True)

        def wait_scatter(p):
            pltpu.make_async_copy(rbufs[p], acc.at[dvs[p]], sss[p]).wait()

        for p in range(3):
            load_src(p, p)
            load_dst(p, p)

        def zrow(i, carry):
            for j in range(d // 16):
                rows2[i, pl.ds(j * 16, 16)] = jnp.zeros((16,), jnp.float32)
            return carry

        lax.fori_loop(0, K, zrow, 0)
        for r in range(rpt // K):
            pltpu.sync_copy(rows2, acc.at[pl.ds(s * rpt + r * K, K)])
        plsc.subcore_barrier()

        for p in range(2):
            wait_src(p)
            issue_gather(p)

        wait_gather(0)
        wait_dst(0)
        issue_scatter(0)
        load_src(3, 0)
        wait_src(2)
        issue_gather(2)

        wait_gather(1)
        wait_dst(1)
        issue_scatter(1)
        load_src(4, 1)
        wait_scatter(0)
        load_dst(3, 0)
        wait_src(0)
        issue_gather(0)

        def chunk_body(cn, p):
            p2 = (p + 2) % 3
            wait_gather(p)
            wait_dst(p)
            issue_scatter(p)

            @pl.when(cn + 3 < steps)
            def _():
                load_src(cn + 3, p)

            @pl.when(cn + 2 < steps)
            def _():
                wait_scatter(p2)
                load_dst(cn + 2, p2)
                wait_src(p2)
                issue_gather(p2)

        def step(j, carry):
            c0 = 3 * j + 2
            chunk_body(c0, 2)
            chunk_body(c0 + 1, 0)
            chunk_body(c0 + 2, 1)
            return carry

        lax.fori_loop(0, (steps - 2) // 3, step, 0)
        for p in ((steps - 3) % 3, (steps - 2) % 3, (steps - 1) % 3):
            wait_scatter(p)
        plsc.subcore_barrier()
        pltpu.sync_copy(acc.at[pl.ds(s * rpt, rpt)],
                        out_hbm.at[c, pl.ds(s * rpt, rpt)])

    return k(src_p, dst_p, y)


def kernel(edge_index, entity_embeddings, W1, b1, W2, b2, Wc, bc):
    src = edge_index[0]
    dst = edge_index[1]
    n = entity_embeddings.shape[0]
    y1 = _dense_first(entity_embeddings, W1, b1)
    p1 = _scatter_partials(y1, src, dst)
    y2 = _dense_mid(p1, W2, b2, n)
    p2 = _scatter_partials(y2, src, dst)
    return _classifier(p2, Wc, bc, n)
